# baseline (device time: 58200 ns/iter reference)
import functools

import jax
import jax.numpy as jnp
from jax import lax
from jax.experimental import pallas as pl
from jax.experimental.pallas import tpu as pltpu

N_Z = 4
N_REP = 4

_SEND_ORDER = {0: (1, 2, 3), 1: (1, 2, 3), 2: (1, 3, 2), 3: (3, 2, 1)}
_WAIT_ORDER = {0: (1, 2, 3), 1: (3, 1, 2), 2: (1, 3, 2), 3: (3, 2, 1)}


def kernel(x):
    m, n_full = x.shape
    n = n_full // N_Z
    qm = m // N_REP
    hq = qm // 2

    def body(x_ref, out_ref, z_send, z_recv, xd_send, xd_recv,
             yd_send, yd_recv, xr_send, xr_recv, yr_send, yr_recv):
        my_x = lax.axis_index("x")
        my_y = lax.axis_index("y")
        my_z = lax.axis_index("z")
        r_me = 2 * my_x + my_y
        r_xn = 2 * (1 - my_x) + my_y
        r_yn = 2 * my_x + (1 - my_y)
        x_nbr = (1 - my_x, my_y)
        y_nbr = (my_x, 1 - my_y)

        barrier_sem = pltpu.get_barrier_semaphore()
        for d in range(1, N_Z):
            q = lax.rem(my_z + d, N_Z)
            pl.semaphore_signal(
                barrier_sem, inc=1,
                device_id=(my_x, my_y, q),
                device_id_type=pl.DeviceIdType.MESH,
            )
        for dev in (x_nbr, y_nbr):
            pl.semaphore_signal(
                barrier_sem, inc=1,
                device_id=dev + (my_z,),
                device_id_type=pl.DeviceIdType.MESH,
            )
        pl.semaphore_wait(barrier_sem, 5)

        out_ref[pl.ds(my_z * m, m), :] = x_ref[:, pl.ds(my_z * n, n)]

        def exchange(Z):
            zr = {}
            for d in _SEND_ORDER[Z]:
                q = (Z + d) % N_Z
                rdma = pltpu.make_async_remote_copy(
                    src_ref=x_ref.at[pl.ds(r_me * qm, qm), pl.ds(q * n, n)],
                    dst_ref=out_ref.at[pl.ds(Z * m + r_me * qm, qm), :],
                    send_sem=z_send.at[d - 1],
                    recv_sem=z_recv.at[(N_Z - 1) - d],
                    device_id=(my_x, my_y, q),
                    device_id_type=pl.DeviceIdType.MESH,
                )
                rdma.start()
                zr[d] = rdma

            xds, yds = {}, {}
            for d in _WAIT_ORDER[Z]:
                zr[d].wait_recv()
                p = (Z + d) % N_Z
                s = (N_Z - 1) - d
                rows = out_ref.at[pl.ds(p * m + r_me * qm, qm), :]
                for dev, ssem, rsem, acc in (
                    (x_nbr, xd_send, xd_recv, xds),
                    (y_nbr, yd_send, yd_recv, yds),
                ):
                    fwd = pltpu.make_async_remote_copy(
                        src_ref=rows,
                        dst_ref=rows,
                        send_sem=ssem.at[s],
                        recv_sem=rsem.at[s],
                        device_id=dev + (Z,),
                        device_id_type=pl.DeviceIdType.MESH,
                    )
                    fwd.start()
                    acc[d] = fwd

            xrs, yrs = [], []
            for d in _WAIT_ORDER[Z]:
                p = (Z + d) % N_Z
                s = (N_Z - 1) - d
                xds[d].wait_recv()
                rows = out_ref.at[pl.ds(p * m + r_xn * qm, hq), :]
                yr = pltpu.make_async_remote_copy(
                    src_ref=rows,
                    dst_ref=rows,
                    send_sem=yr_send.at[s],
                    recv_sem=yr_recv.at[s],
                    device_id=y_nbr + (Z,),
                    device_id_type=pl.DeviceIdType.MESH,
                )
                yr.start()
                yrs.append(yr)

                yds[d].wait_recv()
                rows = out_ref.at[pl.ds(p * m + r_yn * qm + hq, hq), :]
                xr = pltpu.make_async_remote_copy(
                    src_ref=rows,
                    dst_ref=rows,
                    send_sem=xr_send.at[s],
                    recv_sem=xr_recv.at[s],
                    device_id=x_nbr + (Z,),
                    device_id_type=pl.DeviceIdType.MESH,
                )
                xr.start()
                xrs.append(xr)

            for rel in xrs + yrs:
                rel.wait_recv()

            for d in range(1, N_Z):
                zr[d].wait_send()
            for fwd in list(xds.values()) + list(yds.values()) + xrs + yrs:
                fwd.wait_send()

        for Z in range(N_Z):
            pl.when(my_z == Z)(functools.partial(exchange, Z))

    return pl.pallas_call(
        body,
        out_shape=jax.ShapeDtypeStruct((N_Z * m, n), x.dtype),
        in_specs=[pl.BlockSpec(memory_space=pltpu.VMEM)],
        out_specs=pl.BlockSpec(memory_space=pltpu.VMEM),
        scratch_shapes=[
            pltpu.SemaphoreType.DMA((N_Z - 1,)),
            pltpu.SemaphoreType.DMA((N_Z - 1,)),
            pltpu.SemaphoreType.DMA((N_Z - 1,)),
            pltpu.SemaphoreType.DMA((N_Z - 1,)),
            pltpu.SemaphoreType.DMA((N_Z - 1,)),
            pltpu.SemaphoreType.DMA((N_Z - 1,)),
            pltpu.SemaphoreType.DMA((N_Z - 1,)),
            pltpu.SemaphoreType.DMA((N_Z - 1,)),
            pltpu.SemaphoreType.DMA((N_Z - 1,)),
            pltpu.SemaphoreType.DMA((N_Z - 1,)),
        ],
        compiler_params=pltpu.CompilerParams(collective_id=0),
    )(x)


# device time: 49703 ns/iter; 1.1710x vs baseline; 1.1710x over previous
import jax
import jax.numpy as jnp
from jax import lax
from jax.experimental import pallas as pl
from jax.experimental.pallas import tpu as pltpu

N_Z = 4
N_REP = 4


def kernel(x):
    m, n_full = x.shape
    n = n_full // N_Z
    qm = m // N_REP
    hq = qm // 2

    def body(x_ref, out_ref, z_send, z_recv, xd_send, xd_recv,
             yd_send, yd_recv, xr_send, xr_recv, yr_send, yr_recv):
        my_x = lax.axis_index("x")
        my_y = lax.axis_index("y")
        my_z = lax.axis_index("z")
        r_me = 2 * my_x + my_y
        r_xn = 2 * (1 - my_x) + my_y
        r_yn = 2 * my_x + (1 - my_y)
        x_nbr = (1 - my_x, my_y, my_z)
        y_nbr = (my_x, 1 - my_y, my_z)

        barrier_sem = pltpu.get_barrier_semaphore()
        for d in range(1, N_Z):
            q = lax.rem(my_z + d, N_Z)
            pl.semaphore_signal(
                barrier_sem, inc=1,
                device_id=(my_x, my_y, q),
                device_id_type=pl.DeviceIdType.MESH,
            )
        for dev in (x_nbr, y_nbr):
            pl.semaphore_signal(
                barrier_sem, inc=1,
                device_id=dev,
                device_id_type=pl.DeviceIdType.MESH,
            )
        pl.semaphore_wait(barrier_sem, 5)

        z_rdmas = {}
        for d in range(1, N_Z):
            q = lax.rem(my_z + d, N_Z)
            for h in range(2):
                rdma = pltpu.make_async_remote_copy(
                    src_ref=x_ref.at[pl.ds(r_me * qm + h * hq, hq),
                                     pl.ds(q * n, n)],
                    dst_ref=out_ref.at[pl.ds(my_z * m + r_me * qm + h * hq,
                                             hq), :],
                    send_sem=z_send.at[2 * (d - 1) + h],
                    recv_sem=z_recv.at[2 * ((N_Z - 1) - d) + h],
                    device_id=(my_x, my_y, q),
                    device_id_type=pl.DeviceIdType.MESH,
                )
                rdma.start()
                z_rdmas[(d, h)] = rdma

        out_ref[pl.ds(my_z * m, m), :] = x_ref[:, pl.ds(my_z * n, n)]

        xds, yds = {}, {}
        for s in range(N_Z - 1):
            p = lax.rem(my_z + s + 1, N_Z)
            for h in range(2):
                z_rdmas[((N_Z - 1) - s, h)].wait_recv()
                rows = out_ref.at[pl.ds(p * m + r_me * qm + h * hq, hq), :]
                for dev, ssem, rsem, acc in (
                    (x_nbr, xd_send, xd_recv, xds),
                    (y_nbr, yd_send, yd_recv, yds),
                ):
                    fwd = pltpu.make_async_remote_copy(
                        src_ref=rows,
                        dst_ref=rows,
                        send_sem=ssem.at[2 * s + h],
                        recv_sem=rsem.at[2 * s + h],
                        device_id=dev,
                        device_id_type=pl.DeviceIdType.MESH,
                    )
                    fwd.start()
                    acc[(s, h)] = fwd

        xrs, yrs = [], []
        for s in range(N_Z - 1):
            p = lax.rem(my_z + s + 1, N_Z)
            xds[(s, 0)].wait_recv()
            rows = out_ref.at[pl.ds(p * m + r_xn * qm, hq), :]
            yr = pltpu.make_async_remote_copy(
                src_ref=rows,
                dst_ref=rows,
                send_sem=yr_send.at[s],
                recv_sem=yr_recv.at[s],
                device_id=y_nbr,
                device_id_type=pl.DeviceIdType.MESH,
            )
            yr.start()
            yrs.append(yr)

            yds[(s, 1)].wait_recv()
            rows = out_ref.at[pl.ds(p * m + r_yn * qm + hq, hq), :]
            xr = pltpu.make_async_remote_copy(
                src_ref=rows,
                dst_ref=rows,
                send_sem=xr_send.at[s],
                recv_sem=xr_recv.at[s],
                device_id=x_nbr,
                device_id_type=pl.DeviceIdType.MESH,
            )
            xr.start()
            xrs.append(xr)

        for s in range(N_Z - 1):
            xds[(s, 1)].wait_recv()
            yds[(s, 0)].wait_recv()
        for s in range(N_Z - 1):
            xrs[s].wait_recv()
            yrs[s].wait_recv()

        for rdma in z_rdmas.values():
            rdma.wait_send()
        for fwd in list(xds.values()) + list(yds.values()) + xrs + yrs:
            fwd.wait_send()

    return pl.pallas_call(
        body,
        out_shape=jax.ShapeDtypeStruct((N_Z * m, n), x.dtype),
        in_specs=[pl.BlockSpec(memory_space=pltpu.VMEM)],
        out_specs=pl.BlockSpec(memory_space=pltpu.VMEM),
        scratch_shapes=[
            pltpu.SemaphoreType.DMA((2 * (N_Z - 1),)),
            pltpu.SemaphoreType.DMA((2 * (N_Z - 1),)),
            pltpu.SemaphoreType.DMA((2 * (N_Z - 1),)),
            pltpu.SemaphoreType.DMA((2 * (N_Z - 1),)),
            pltpu.SemaphoreType.DMA((2 * (N_Z - 1),)),
            pltpu.SemaphoreType.DMA((2 * (N_Z - 1),)),
            pltpu.SemaphoreType.DMA((N_Z - 1,)),
            pltpu.SemaphoreType.DMA((N_Z - 1,)),
            pltpu.SemaphoreType.DMA((N_Z - 1,)),
            pltpu.SemaphoreType.DMA((N_Z - 1,)),
        ],
        compiler_params=pltpu.CompilerParams(collective_id=0),
    )(x)


# device time: 48776 ns/iter; 1.1932x vs baseline; 1.0190x over previous
import functools

import jax
import jax.numpy as jnp
from jax import lax
from jax.experimental import pallas as pl
from jax.experimental.pallas import tpu as pltpu

N_Z = 4
N_REP = 4

_WAIT_ORDER = {0: (1, 2, 3), 1: (3, 1, 2), 2: (3, 1, 2), 3: (3, 2, 1)}


def kernel(x):
    m, n_full = x.shape
    n = n_full // N_Z
    qm = m // N_REP
    hq = qm // 2

    def body(x_ref, out_ref, z_send, z_recv, xd_send, xd_recv,
             yd_send, yd_recv, xr_send, xr_recv, yr_send, yr_recv):
        my_x = lax.axis_index("x")
        my_y = lax.axis_index("y")
        my_z = lax.axis_index("z")
        r_me = 2 * my_x + my_y
        r_xn = 2 * (1 - my_x) + my_y
        r_yn = 2 * my_x + (1 - my_y)
        r_gn = 2 * (1 - my_x) + (1 - my_y)
        x_nbr = (1 - my_x, my_y, my_z)
        y_nbr = (my_x, 1 - my_y, my_z)

        barrier_sem = pltpu.get_barrier_semaphore()
        for d in range(1, N_Z):
            q = lax.rem(my_z + d, N_Z)
            pl.semaphore_signal(
                barrier_sem, inc=1,
                device_id=(my_x, my_y, q),
                device_id_type=pl.DeviceIdType.MESH,
            )
        for dev in (x_nbr, y_nbr):
            pl.semaphore_signal(
                barrier_sem, inc=1,
                device_id=dev,
                device_id_type=pl.DeviceIdType.MESH,
            )
        pl.semaphore_wait(barrier_sem, 5)

        z_rdmas = {}
        for d in range(1, N_Z):
            q = lax.rem(my_z + d, N_Z)
            for h in range(2):
                rdma = pltpu.make_async_remote_copy(
                    src_ref=x_ref.at[pl.ds(r_me * qm + h * hq, hq),
                                     pl.ds(q * n, n)],
                    dst_ref=out_ref.at[pl.ds(my_z * m + r_me * qm + h * hq,
                                             hq), :],
                    send_sem=z_send.at[2 * (d - 1) + h],
                    recv_sem=z_recv.at[2 * ((N_Z - 1) - d) + h],
                    device_id=(my_x, my_y, q),
                    device_id_type=pl.DeviceIdType.MESH,
                )
                rdma.start()
                z_rdmas[(d, h)] = rdma

        out_ref[pl.ds(my_z * m, m), :] = x_ref[:, pl.ds(my_z * n, n)]

        def spread(Z):
            xds, yds = {}, {}
            for d in _WAIT_ORDER[Z]:
                p = (Z + d) % N_Z
                s = (N_Z - 1) - d
                for h in range(2):
                    z_rdmas[(d, h)].wait_recv()
                    rows = out_ref.at[pl.ds(p * m + r_me * qm + h * hq,
                                            hq), :]
                    for dev, ssem, rsem, acc in (
                        (x_nbr, xd_send, xd_recv, xds),
                        (y_nbr, yd_send, yd_recv, yds),
                    ):
                        fwd = pltpu.make_async_remote_copy(
                            src_ref=rows,
                            dst_ref=rows,
                            send_sem=ssem.at[2 * s + h],
                            recv_sem=rsem.at[2 * s + h],
                            device_id=dev,
                            device_id_type=pl.DeviceIdType.MESH,
                        )
                        fwd.start()
                        acc[(s, h)] = fwd

            for d in _WAIT_ORDER[Z]:
                p = (Z + d) % N_Z
                s = (N_Z - 1) - d
                xds[(s, 0)].wait_recv()
                rows = out_ref.at[pl.ds(p * m + r_xn * qm, hq), :]
                yr = pltpu.make_async_remote_copy(
                    src_ref=rows,
                    dst_ref=rows,
                    send_sem=yr_send.at[s],
                    recv_sem=yr_recv.at[s],
                    device_id=y_nbr,
                    device_id_type=pl.DeviceIdType.MESH,
                )
                yr.start()

                yds[(s, 1)].wait_recv()
                rows = out_ref.at[pl.ds(p * m + r_yn * qm + hq, hq), :]
                xr = pltpu.make_async_remote_copy(
                    src_ref=rows,
                    dst_ref=rows,
                    send_sem=xr_send.at[s],
                    recv_sem=xr_recv.at[s],
                    device_id=x_nbr,
                    device_id_type=pl.DeviceIdType.MESH,
                )
                xr.start()

        for Z in range(N_Z):
            pl.when(my_z == Z)(functools.partial(spread, Z))

        def waiter(rows_ref, ssem, rsem):
            return pltpu.make_async_remote_copy(
                src_ref=rows_ref, dst_ref=rows_ref,
                send_sem=ssem, recv_sem=rsem,
                device_id=x_nbr, device_id_type=pl.DeviceIdType.MESH,
            )

        for s in range(N_Z - 1):
            p = lax.rem(my_z + s + 1, N_Z)
            waiter(out_ref.at[pl.ds(p * m + r_xn * qm + hq, hq), :],
                   xd_send.at[2 * s + 1], xd_recv.at[2 * s + 1]).wait_recv()
            waiter(out_ref.at[pl.ds(p * m + r_yn * qm, hq), :],
                   yd_send.at[2 * s], yd_recv.at[2 * s]).wait_recv()
            waiter(out_ref.at[pl.ds(p * m + r_gn * qm + hq, hq), :],
                   xr_send.at[s], xr_recv.at[s]).wait_recv()
            waiter(out_ref.at[pl.ds(p * m + r_gn * qm, hq), :],
                   yr_send.at[s], yr_recv.at[s]).wait_recv()

        for rdma in z_rdmas.values():
            rdma.wait_send()
        for s in range(N_Z - 1):
            p = lax.rem(my_z + s + 1, N_Z)
            for h in range(2):
                rows = out_ref.at[pl.ds(p * m + r_me * qm + h * hq, hq), :]
                waiter(rows, xd_send.at[2 * s + h],
                       xd_recv.at[2 * s + h]).wait_send()
                waiter(rows, yd_send.at[2 * s + h],
                       yd_recv.at[2 * s + h]).wait_send()
            waiter(out_ref.at[pl.ds(p * m + r_xn * qm, hq), :],
                   yr_send.at[s], yr_recv.at[s]).wait_send()
            waiter(out_ref.at[pl.ds(p * m + r_yn * qm + hq, hq), :],
                   xr_send.at[s], xr_recv.at[s]).wait_send()

    return pl.pallas_call(
        body,
        out_shape=jax.ShapeDtypeStruct((N_Z * m, n), x.dtype),
        in_specs=[pl.BlockSpec(memory_space=pltpu.VMEM)],
        out_specs=pl.BlockSpec(memory_space=pltpu.VMEM),
        scratch_shapes=[
            pltpu.SemaphoreType.DMA((2 * (N_Z - 1),)),
            pltpu.SemaphoreType.DMA((2 * (N_Z - 1),)),
            pltpu.SemaphoreType.DMA((2 * (N_Z - 1),)),
            pltpu.SemaphoreType.DMA((2 * (N_Z - 1),)),
            pltpu.SemaphoreType.DMA((2 * (N_Z - 1),)),
            pltpu.SemaphoreType.DMA((2 * (N_Z - 1),)),
            pltpu.SemaphoreType.DMA((N_Z - 1,)),
            pltpu.SemaphoreType.DMA((N_Z - 1,)),
            pltpu.SemaphoreType.DMA((N_Z - 1,)),
            pltpu.SemaphoreType.DMA((N_Z - 1,)),
        ],
        compiler_params=pltpu.CompilerParams(collective_id=0),
    )(x)
